# baseline (speedup 1.0000x reference)
import dataclasses
import functools

import jax
import jax.numpy as jnp
from jax import lax
from jax.experimental import pallas as pl
from jax.experimental.pallas import tpu as pltpu
from jax.experimental.pallas import tpu_sc as plsc

_B = 16384
_V = 100000
_E = 128
_NC = 2
_NS = 16
_NW = _NC * _NS
_BPW = _B // _NW
_CH = 64
_NCH = _BPW // _CH
_L = 16


def _sc_body(ti, tj, pi, pj, bi, bj, t_o,
             idx_i, idx_j, ri0, rj0, ri1, rj1, part_v, score_v, bib, bjb,
             sem_i0, sem_j0, sem_i1, sem_j1, sem_bi, sem_bj):
    wid = lax.axis_index("s") * _NC + lax.axis_index("c")
    base = wid * _BPW
    cpi = pltpu.async_copy(pi.at[pl.ds(base, _BPW)], idx_i, sem_i0)
    cpj = pltpu.async_copy(pj.at[pl.ds(base, _BPW)], idx_j, sem_j0)
    cpi.wait()
    cpj.wait()

    bias_copies = []
    for c in range(_NCH):
        sl = pl.ds(c * _CH, _CH)
        bias_copies.append(pltpu.async_copy(bi.at[idx_i.at[sl]], bib.at[sl], sem_bi))
        bias_copies.append(pltpu.async_copy(bj.at[idx_j.at[sl]], bjb.at[sl], sem_bj))

    rbufs = ((ri0, rj0, sem_i0, sem_j0), (ri1, rj1, sem_i1, sem_j1))

    def issue(c):
        sl = pl.ds(c * _CH, _CH)
        ri, rj, si, sj = rbufs[c % 2]
        return (pltpu.async_copy(ti.at[idx_i.at[sl]], ri, si),
                pltpu.async_copy(tj.at[idx_j.at[sl]], rj, sj))

    lane = lax.iota(jnp.int32, _L)

    inflight = issue(0)
    for c in range(_NCH):
        cp_i, cp_j = inflight
        if c + 1 < _NCH:
            nxt = issue(c + 1)
        cp_i.wait()
        cp_j.wait()
        ri, rj, _, _ = rbufs[c % 2]

        @plsc.parallel_loop(0, _CH, unroll=2)
        def _row(r):
            prods = [ri[r, pl.ds(k * _L, _L)] * rj[r, pl.ds(k * _L, _L)]
                     for k in range(_E // _L)]
            while len(prods) > 1:
                prods = [prods[i] + prods[i + 1]
                         for i in range(0, len(prods), 2)]
            part_v[pl.ds(r * _L, _L)] = prods[0]

        @plsc.parallel_loop(0, _CH // _L, unroll=2)
        def _grp(g):
            gbase = g * (_L * _L) + lane * _L
            acc = plsc.load_gather(part_v, [gbase])
            for k in range(1, _L):
                acc = acc + plsc.load_gather(part_v, [gbase + k])
            score_v[pl.ds(c * _CH + g * _L, _L)] = acc

        if c + 1 < _NCH:
            inflight = nxt

    for cp in bias_copies:
        cp.wait()

    @pl.loop(0, _BPW // _L)
    def _addb(g):
        sl = pl.ds(g * _L, _L)
        score_v[sl] = score_v[sl] + bib[sl] + bjb[sl]

    pltpu.sync_copy(score_v, t_o.at[pl.ds(base, _BPW)])


@functools.lru_cache(maxsize=1)
def _sc_gather_dot():
    cp = pltpu.CompilerParams()
    if "needs_layout_passes" in pltpu.CompilerParams.__dataclass_fields__:
        cp = dataclasses.replace(cp, needs_layout_passes=False)
    return pl.kernel(
        _sc_body,
        mesh=plsc.VectorSubcoreMesh(core_axis_name="c", subcore_axis_name="s"),
        compiler_params=cp,
        out_type=jax.ShapeDtypeStruct((_B,), jnp.float32),
        scratch_types=[
            pltpu.VMEM((_BPW,), jnp.int32),
            pltpu.VMEM((_BPW,), jnp.int32),
            pltpu.VMEM((_CH, _E), jnp.float32),
            pltpu.VMEM((_CH, _E), jnp.float32),
            pltpu.VMEM((_CH, _E), jnp.float32),
            pltpu.VMEM((_CH, _E), jnp.float32),
            pltpu.VMEM((_CH * _L,), jnp.float32),
            pltpu.VMEM((_BPW,), jnp.float32),
            pltpu.VMEM((_BPW,), jnp.float32),
            pltpu.VMEM((_BPW,), jnp.float32),
            pltpu.SemaphoreType.DMA,
            pltpu.SemaphoreType.DMA,
            pltpu.SemaphoreType.DMA,
            pltpu.SemaphoreType.DMA,
            pltpu.SemaphoreType.DMA,
            pltpu.SemaphoreType.DMA,
        ],
    )


def _tc_body(t_ref, wij_ref, wf_ref, out_ref):
    d = t_ref[...] - jnp.log(wij_ref[...])
    out_ref[...] = jnp.sum(d * d * wf_ref[...], keepdims=True)


def _loss_tc(t2, wij2, wf2):
    return pl.pallas_call(
        _tc_body,
        out_shape=jax.ShapeDtypeStruct((1, 1), jnp.float32),
    )(t2, wij2, wf2)


def kernel(pos_i, pos_j, wij, wf, input_embs, output_embs, bi_table, bj_table):
    t = _sc_gather_dot()(
        input_embs, output_embs,
        pos_i.reshape(_B).astype(jnp.int32), pos_j.reshape(_B).astype(jnp.int32),
        bi_table.reshape(_V), bj_table.reshape(_V))
    sq = _B // 128
    out = _loss_tc(t.reshape(sq, 128), wij.reshape(sq, 128),
                   wf.reshape(sq, 128))
    return out.reshape(())


# speedup vs baseline: 1.0119x; 1.0119x over previous
import dataclasses
import functools

import jax
import jax.numpy as jnp
from jax import lax
from jax.experimental import pallas as pl
from jax.experimental.pallas import tpu as pltpu
from jax.experimental.pallas import tpu_sc as plsc

_B = 16384
_V = 100000
_E = 128
_NC = 2
_NS = 16
_NW = _NC * _NS
_BPW = _B // _NW
_CH = 128
_NCH = _BPW // _CH
_L = 16


def _sc_body(ti, tj, pi, pj, bi, bj, t_o,
             idx_i, idx_j, ri0, rj0, ri1, rj1, part_v, score_v, bib, bjb,
             sem_i0, sem_j0, sem_i1, sem_j1, sem_bi, sem_bj):
    wid = lax.axis_index("s") * _NC + lax.axis_index("c")
    base = wid * _BPW
    cpi = pltpu.async_copy(pi.at[pl.ds(base, _BPW)], idx_i, sem_i0)
    cpj = pltpu.async_copy(pj.at[pl.ds(base, _BPW)], idx_j, sem_j0)
    cpi.wait()
    cpj.wait()

    bias_copies = []
    for c in range(_NCH):
        sl = pl.ds(c * _CH, _CH)
        bias_copies.append(pltpu.async_copy(bi.at[idx_i.at[sl]], bib.at[sl], sem_bi))
        bias_copies.append(pltpu.async_copy(bj.at[idx_j.at[sl]], bjb.at[sl], sem_bj))

    rbufs = ((ri0, rj0, sem_i0, sem_j0), (ri1, rj1, sem_i1, sem_j1))

    def issue(c):
        sl = pl.ds(c * _CH, _CH)
        ri, rj, si, sj = rbufs[c % 2]
        return (pltpu.async_copy(ti.at[idx_i.at[sl]], ri, si),
                pltpu.async_copy(tj.at[idx_j.at[sl]], rj, sj))

    lane = lax.iota(jnp.int32, _L)

    inflight = issue(0)
    for c in range(_NCH):
        cp_i, cp_j = inflight
        if c + 1 < _NCH:
            nxt = issue(c + 1)
        cp_i.wait()
        cp_j.wait()
        ri, rj, _, _ = rbufs[c % 2]

        @plsc.parallel_loop(0, _CH, unroll=2)
        def _row(r):
            prods = [ri[r, pl.ds(k * _L, _L)] * rj[r, pl.ds(k * _L, _L)]
                     for k in range(_E // _L)]
            while len(prods) > 1:
                prods = [prods[i] + prods[i + 1]
                         for i in range(0, len(prods), 2)]
            part_v[pl.ds(r * _L, _L)] = prods[0]

        @plsc.parallel_loop(0, _CH // _L, unroll=2)
        def _grp(g):
            gbase = g * (_L * _L) + lane * _L
            acc = plsc.load_gather(part_v, [gbase])
            for k in range(1, _L):
                acc = acc + plsc.load_gather(part_v, [gbase + k])
            score_v[pl.ds(c * _CH + g * _L, _L)] = acc

        if c + 1 < _NCH:
            inflight = nxt

    for cp in bias_copies:
        cp.wait()

    @pl.loop(0, _BPW // _L)
    def _addb(g):
        sl = pl.ds(g * _L, _L)
        score_v[sl] = score_v[sl] + bib[sl] + bjb[sl]

    pltpu.sync_copy(score_v, t_o.at[pl.ds(base, _BPW)])


@functools.lru_cache(maxsize=1)
def _sc_gather_dot():
    cp = pltpu.CompilerParams()
    if "needs_layout_passes" in pltpu.CompilerParams.__dataclass_fields__:
        cp = dataclasses.replace(cp, needs_layout_passes=False)
    return pl.kernel(
        _sc_body,
        mesh=plsc.VectorSubcoreMesh(core_axis_name="c", subcore_axis_name="s"),
        compiler_params=cp,
        out_type=jax.ShapeDtypeStruct((_B,), jnp.float32),
        scratch_types=[
            pltpu.VMEM((_BPW,), jnp.int32),
            pltpu.VMEM((_BPW,), jnp.int32),
            pltpu.VMEM((_CH, _E), jnp.float32),
            pltpu.VMEM((_CH, _E), jnp.float32),
            pltpu.VMEM((_CH, _E), jnp.float32),
            pltpu.VMEM((_CH, _E), jnp.float32),
            pltpu.VMEM((_CH * _L,), jnp.float32),
            pltpu.VMEM((_BPW,), jnp.float32),
            pltpu.VMEM((_BPW,), jnp.float32),
            pltpu.VMEM((_BPW,), jnp.float32),
            pltpu.SemaphoreType.DMA,
            pltpu.SemaphoreType.DMA,
            pltpu.SemaphoreType.DMA,
            pltpu.SemaphoreType.DMA,
            pltpu.SemaphoreType.DMA,
            pltpu.SemaphoreType.DMA,
        ],
    )


def _tc_body(t_ref, wij_ref, wf_ref, out_ref):
    d = t_ref[...] - jnp.log(wij_ref[...])
    out_ref[...] = jnp.sum(d * d * wf_ref[...], keepdims=True)


def _loss_tc(t2, wij2, wf2):
    return pl.pallas_call(
        _tc_body,
        out_shape=jax.ShapeDtypeStruct((1, 1), jnp.float32),
    )(t2, wij2, wf2)


def kernel(pos_i, pos_j, wij, wf, input_embs, output_embs, bi_table, bj_table):
    t = _sc_gather_dot()(
        input_embs, output_embs,
        pos_i.reshape(_B).astype(jnp.int32), pos_j.reshape(_B).astype(jnp.int32),
        bi_table.reshape(_V), bj_table.reshape(_V))
    sq = _B // 128
    out = _loss_tc(t.reshape(sq, 128), wij.reshape(sq, 128),
                   wf.reshape(sq, 128))
    return out.reshape(())
